# Initial kernel scaffold; baseline (speedup 1.0000x reference)
import jax
import jax.numpy as jnp
from jax.experimental import pallas as pl


def kernel(x, edge_index, perm, W_enc, b_enc, codebooks):
    raise NotImplementedError("write your pallas kernel here")


# speedup vs baseline: 1.9677x; 1.9677x over previous
import functools

import jax
import jax.numpy as jnp
from jax import lax
from jax.experimental import pallas as pl
from jax.experimental.pallas import tpu as pltpu
from jax.experimental.pallas import tpu_sc as plsc

N = 10000
E = 160000
D_IN = 256
H = 512
Q = 4
K = 1024

NC = 2
NS = 16
DH = D_IN // NC
ECS = 80
E_PAD = NS * ECS * 128
NPAD = 10240
TRASH = N
ROWS_PER_TILE = NPAD // NS

BN = 512
NB = NPAD // BN


def _sc_body(use_perm, do_deg,
             xcat_hbm, src_hbm, dst_hbm, permr_hbm,
             *refs):
    if do_deg:
        agg2_hbm, deg2_hbm = refs[:2]
        scratches = refs[2:]
        y2_hbm = None
    else:
        agg2_hbm, y2_hbm = refs[:2]
        scratches = refs[2:]
    if use_perm:
        (acc_sp, srcv, dstv, rowsv, zerov, permv5, sem) = scratches
    else:
        (acc_sp, srcv, dstv, rowsv, zerov, sem) = scratches

    c = lax.axis_index("c")
    s = lax.axis_index("s")

    zero16 = jnp.zeros((16,), jnp.float32)
    one16 = jnp.ones((16,), jnp.float32)

    def _zf(i, carry):
        zerov[i // 8, pl.ds((i % 8) * 16, 16)] = zero16
        return carry
    lax.fori_loop(0, 8 * 8, _zf, 0)

    base = s * ROWS_PER_TILE

    def _zacc(k, carry):
        pltpu.sync_copy(zerov, acc_sp.at[pl.ds(base + k * 8, 8)])
        return carry
    lax.fori_loop(0, ROWS_PER_TILE // 8, _zacc, 0)

    pltpu.sync_copy(src_hbm.at[s], srcv)
    pltpu.sync_copy(dst_hbm.at[s], dstv)

    if use_perm:
        pltpu.sync_copy(permr_hbm.at[s], permv5)
        for p in range(2):
            qd = 2 * c + p
            for k in range(ROWS_PER_TILE // 128):
                pltpu.async_copy(xcat_hbm.at[qd].at[permv5.at[k]], rowsv,
                                 sem).wait()
                pltpu.sync_copy(rowsv,
                                y2_hbm.at[qd].at[pl.ds(base + k * 128, 128)])

    plsc.subcore_barrier()

    table_hbm = y2_hbm if use_perm else xcat_hbm

    def _step_for(qd):
        def _step(j, carry):
            pltpu.async_copy(table_hbm.at[qd].at[srcv.at[j]], rowsv,
                             sem).wait()
            pltpu.sync_copy(rowsv, acc_sp.at[dstv.at[j]], add=True)
            return carry
        return _step

    for p in range(2):
        qd = 2 * c + p
        if p > 0:
            plsc.subcore_barrier()
            lax.fori_loop(0, ROWS_PER_TILE // 8, _zacc, 0)
            plsc.subcore_barrier()
        lax.fori_loop(0, ECS, _step_for(qd), 0)
        plsc.subcore_barrier()
        pltpu.sync_copy(acc_sp.at[pl.ds(base, ROWS_PER_TILE)],
                        agg2_hbm.at[qd].at[pl.ds(base, ROWS_PER_TILE)])

    if do_deg:
        plsc.subcore_barrier()
        lax.fori_loop(0, ROWS_PER_TILE // 8, _zacc, 0)

        def _of(i, carry):
            rowsv[i // 8, pl.ds((i % 8) * 16, 16)] = one16
            return carry
        lax.fori_loop(0, 128 * 8, _of, 0)
        plsc.subcore_barrier()

        def _dstep(j, carry):
            pltpu.sync_copy(rowsv, acc_sp.at[dstv.at[c * (ECS // 2) + j]],
                            add=True)
            return carry
        lax.fori_loop(0, ECS // 2, _dstep, 0)
        plsc.subcore_barrier()

        pltpu.sync_copy(acc_sp.at[pl.ds(base, ROWS_PER_TILE)],
                        deg2_hbm.at[c].at[pl.ds(base, ROWS_PER_TILE)])


def _make_sc_call(use_perm, do_deg):
    mesh = plsc.VectorSubcoreMesh(core_axis_name="c", subcore_axis_name="s",
                                  num_cores=NC, num_subcores=NS)
    out_type = [jax.ShapeDtypeStruct((4, NPAD, 128), jnp.float32)]
    if do_deg:
        out_type.append(jax.ShapeDtypeStruct((NC, NPAD, 128), jnp.float32))
    else:
        out_type.append(jax.ShapeDtypeStruct((4, NPAD, 128), jnp.float32))
    scratch_types = [
        pltpu.VMEM_SHARED((NPAD, DH), jnp.float32),
        pltpu.VMEM((ECS, 128), jnp.int32),
        pltpu.VMEM((ECS, 128), jnp.int32),
        pltpu.VMEM((128, DH), jnp.float32),
        pltpu.VMEM((8, 128), jnp.float32),
    ]
    if use_perm:
        scratch_types.append(
            pltpu.VMEM((ROWS_PER_TILE // 128, 128), jnp.int32))
    scratch_types.append(pltpu.SemaphoreType.DMA)
    return pl.kernel(functools.partial(_sc_body, use_perm, do_deg),
                     out_type=out_type, mesh=mesh,
                     scratch_types=scratch_types)


def _cn_body(cb_ref, out_ref):
    cb = cb_ref[0]
    nrm = jnp.sqrt(jnp.sum(cb * cb, axis=-1, keepdims=True))
    out_ref[0] = cb / (nrm + 1e-8)


_cn_call = pl.pallas_call(
    _cn_body,
    grid=(Q,),
    in_specs=[pl.BlockSpec((1, K, H), lambda q: (q, 0, 0))],
    out_specs=pl.BlockSpec((1, K, H), lambda q: (q, 0, 0)),
    out_shape=jax.ShapeDtypeStruct((Q, K, H), jnp.float32),
)


def _mat_body(x_ref, w_ref, m_ref):
    m = jax.lax.dot_general(x_ref[...], w_ref[...], (((1,), (0,)), ((), ())),
                            preferred_element_type=jnp.float32)
    for qd in range(4):
        m_ref[qd] = m[:, qd * 128:(qd + 1) * 128]


_mat_call = pl.pallas_call(
    _mat_body,
    grid=(NB,),
    in_specs=[
        pl.BlockSpec((BN, D_IN), lambda i: (i, 0)),
        pl.BlockSpec((D_IN, H), lambda i: (0, 0)),
    ],
    out_specs=pl.BlockSpec((4, BN, 128), lambda i: (0, i, 0)),
    out_shape=jax.ShapeDtypeStruct((4, NPAD, 128), jnp.float32),
)


def _tc_body(a0_ref, a1_ref, a2_ref, a3_ref, dega_ref, degb_ref, b_ref,
             cn_ref,
             z_ref, q_ref, idx_ref, s_ref, sq_ref, loss_ref,
             zsum, qsum, lsum):
    i = pl.program_id(0)

    @pl.when(i == 0)
    def _init():
        zsum[...] = jnp.zeros_like(zsum)
        qsum[...] = jnp.zeros_like(qsum)
        for qi in range(Q):
            lsum[qi] = 0.0

    agg = jnp.concatenate([a0_ref[...], a1_ref[...], a2_ref[...],
                           a3_ref[...]], axis=1)
    deg = jnp.maximum(dega_ref[:, 0:1] + degb_ref[:, 0:1], 1.0)
    z = jnp.maximum(agg / deg + b_ref[...], 0.0)
    z_ref[...] = z

    rowid = i * BN + lax.broadcasted_iota(jnp.int32, (BN, 1), 0)
    mask = (rowid < N).astype(jnp.float32)

    r = z
    qacc = jnp.zeros_like(z)
    idx_cols = []
    for qi in range(Q):
        cn = cn_ref[qi]
        rn = r / (jnp.sqrt(jnp.sum(r * r, axis=-1, keepdims=True)) + 1e-8)
        sim = jax.lax.dot_general(rn, cn, (((1,), (1,)), ((), ())),
                                  preferred_element_type=jnp.float32)
        mx = jnp.max(sim, axis=-1, keepdims=True)
        kiota = lax.broadcasted_iota(jnp.int32, (BN, K), 1)
        idx = jnp.min(jnp.where(sim >= mx, kiota, K), axis=-1)
        onehot = (kiota == idx[:, None]).astype(jnp.float32)
        quant = jax.lax.dot_general(onehot, cn, (((1,), (0,)), ((), ())),
                                    precision=lax.Precision.HIGHEST,
                                    preferred_element_type=jnp.float32)
        diff = quant - r
        lsum[qi] += jnp.sum(diff * diff * mask)
        idx_cols.append(idx[:, None])
        r = r - quant
        qacc = qacc + quant

    q_ref[...] = qacc
    idx_ref[...] = jnp.concatenate(idx_cols, axis=1)
    zsum[...] += jnp.sum(z * mask, axis=0, keepdims=True)
    qsum[...] += jnp.sum(qacc * mask, axis=0, keepdims=True)

    @pl.when(i == NB - 1)
    def _fin():
        s_ref[...] = jax.nn.sigmoid(zsum[...] / N)
        sq_ref[...] = jax.nn.sigmoid(qsum[...] / N)
        total = (lsum[0] + lsum[1] + lsum[2] + lsum[3]) / (N * H)
        loss_ref[...] = jnp.full((1, 1), total, jnp.float32)


_tc_call = pl.pallas_call(
    _tc_body,
    grid=(NB,),
    in_specs=[
        pl.BlockSpec((BN, 128), lambda i: (i, 0)),
        pl.BlockSpec((BN, 128), lambda i: (i, 0)),
        pl.BlockSpec((BN, 128), lambda i: (i, 0)),
        pl.BlockSpec((BN, 128), lambda i: (i, 0)),
        pl.BlockSpec((BN, 128), lambda i: (i, 0)),
        pl.BlockSpec((BN, 128), lambda i: (i, 0)),
        pl.BlockSpec((1, H), lambda i: (0, 0)),
        pl.BlockSpec((Q, K, H), lambda i: (0, 0, 0)),
    ],
    out_specs=[
        pl.BlockSpec((BN, H), lambda i: (i, 0)),
        pl.BlockSpec((BN, H), lambda i: (i, 0)),
        pl.BlockSpec((BN, Q), lambda i: (i, 0)),
        pl.BlockSpec((1, H), lambda i: (0, 0)),
        pl.BlockSpec((1, H), lambda i: (0, 0)),
        pl.BlockSpec((1, 1), lambda i: (0, 0)),
    ],
    out_shape=[
        jax.ShapeDtypeStruct((N, H), jnp.float32),
        jax.ShapeDtypeStruct((N, H), jnp.float32),
        jax.ShapeDtypeStruct((N, Q), jnp.int32),
        jax.ShapeDtypeStruct((1, H), jnp.float32),
        jax.ShapeDtypeStruct((1, H), jnp.float32),
        jax.ShapeDtypeStruct((1, 1), jnp.float32),
    ],
    scratch_shapes=[
        pltpu.VMEM((1, H), jnp.float32),
        pltpu.VMEM((1, H), jnp.float32),
        pltpu.SMEM((Q,), jnp.float32),
    ],
)


def kernel(x, edge_index, perm, W_enc, b_enc, codebooks):
    src = edge_index[0]
    dst = edge_index[1]
    pad = E_PAD - E
    srcp = jnp.concatenate([src, jnp.zeros((pad,), jnp.int32)]).reshape(
        NS, ECS, 128)
    dstp = jnp.concatenate([dst, jnp.full((pad,), TRASH, jnp.int32)]).reshape(
        NS, ECS, 128)
    permr = jnp.concatenate([perm, jnp.zeros((NPAD - N,), jnp.int32)]).reshape(
        NS, ROWS_PER_TILE // 128, 128)
    xpad = jnp.concatenate(
        [x, jnp.zeros((NPAD - N, D_IN), jnp.float32)], axis=0)

    m4 = _mat_call(xpad, W_enc)
    agg4_p, deg2 = _make_sc_call(False, True)(m4, srcp, dstp, permr)
    agg4_n, _ = _make_sc_call(True, False)(m4, srcp, dstp, permr)

    cn = _cn_call(codebooks)
    b2 = b_enc.reshape(1, H)

    zp, qp, idxp, sp, sqp, lp = _tc_call(
        agg4_p[0], agg4_p[1], agg4_p[2], agg4_p[3],
        deg2[0], deg2[1], b2, cn)
    zn, qn, _, _, _, ln = _tc_call(
        agg4_n[0], agg4_n[1], agg4_n[2], agg4_n[3],
        deg2[0], deg2[1], b2, cn)

    return (zp, zn, sp[0], qp, qn, sqp[0], lp[0, 0], ln[0, 0], idxp)
